# baseline (speedup 1.0000x reference)
import functools

import jax
import jax.numpy as jnp
from jax import lax
from jax.experimental import pallas as pl
from jax.experimental.pallas import tpu as pltpu
from jax.experimental.pallas import tpu_sc as plsc

B = 16384
D = 64
NW = 32
BPW = B // NW
PPW = BPW // 2
CHUNK = 32
NCHUNK = BPW // CHUNK
NTOT = B * D


def _sc_gather_stats(c2, noise2, table, lw16, ct16):
    mesh = plsc.VectorSubcoreMesh(core_axis_name="c", subcore_axis_name="s")

    @functools.partial(
        pl.kernel,
        mesh=mesh,
        out_type=[
            jax.ShapeDtypeStruct((B // 2, 2 * D), jnp.float32),
            jax.ShapeDtypeStruct((2 * NW, 16), jnp.float32),
        ],
        scratch_types=[
            pltpu.VMEM((BPW,), jnp.int32),
            pltpu.VMEM((BPW + 16,), jnp.int32),
            pltpu.VMEM((BPW + 16,), jnp.int32),
            pltpu.VMEM((CHUNK, 8, D), jnp.float32),
            pltpu.VMEM((CHUNK, 8, D), jnp.float32),
            pltpu.VMEM((PPW, 2 * D), jnp.float32),
            pltpu.VMEM((16,), jnp.float32),
            pltpu.VMEM((16,), jnp.int32),
            pltpu.VMEM((16,), jnp.float32),
            pltpu.VMEM((16,), jnp.float32),
            pltpu.SemaphoreType.DMA,
            pltpu.SemaphoreType.DMA,
            pltpu.SemaphoreType.DMA,
        ],
    )
    def k(c_hbm, noise_hbm, table_hbm, lw_hbm, ct_hbm, y_hbm, part_hbm,
          raw_v, rb_v, row_v, buf0, buf1, noise_v, lw_v, ct_v, s_v, q_v,
          sem0, sem1, nsem):
        bufs = (buf0, buf1)
        sems = (sem0, sem1)
        wid = lax.axis_index("s") * 2 + lax.axis_index("c")
        pltpu.sync_copy(c_hbm.at[wid], raw_v)
        ncopy = pltpu.async_copy(noise_hbm.at[pl.ds(wid * PPW, PPW)], noise_v, nsem)
        for t in range(BPW // 16):
            v = raw_v[pl.ds(t * 16, 16)]
            rb_v[pl.ds(t * 16, 16)] = v >> 3
            row_v[pl.ds(t * 16, 16)] = v & 7

        def fire(k_idx, buf, sem):
            for g in range(0, CHUNK, 16):
                v = rb_v[pl.ds(k_idx * CHUNK + g, 16)]
                for l in range(16):
                    pltpu.async_copy(table_hbm.at[v[l]], buf.at[g + l], sem)

        def drain(buf, sem):
            pltpu.make_async_copy(table_hbm.at[pl.ds(0, CHUNK)], buf, sem).wait()

        pltpu.sync_copy(lw_hbm, lw_v)
        pltpu.sync_copy(ct_hbm, ct_v)
        lw = lw_v[...]
        ct = ct_v[...]
        scale = jnp.where(ct != 0, 1.0 / (1.0 + jnp.exp(-lw)), 0.0)

        fire(0, bufs[0], sems[0])
        ncopy.wait()
        s = jnp.zeros((16,), jnp.float32)
        q = jnp.zeros((16,), jnp.float32)
        for kk in range(NCHUNK):
            if kk + 1 < NCHUNK:
                fire(kk + 1, bufs[(kk + 1) & 1], sems[(kk + 1) & 1])
            drain(bufs[kk & 1], sems[kk & 1])
            buf = bufs[kk & 1]

            def body(p, carry, kk=kk, buf=buf):
                s, q = carry
                rows = row_v[pl.ds(kk * CHUNK + 2 * p, 16)]
                for r in range(2):
                    row = rows[r]
                    for j in range(D // 16):
                        e = buf[2 * p + r, row, pl.ds(j * 16, 16)]
                        nz = noise_v[kk * (CHUNK // 2) + p, pl.ds(r * D + j * 16, 16)]
                        y = e * scale + nz
                        noise_v[kk * (CHUNK // 2) + p, pl.ds(r * D + j * 16, 16)] = y
                        s = s + y
                        q = q + y * y
                return s, q

            s, q = lax.fori_loop(0, CHUNK // 2, body, (s, q))

        s_v[...] = s
        q_v[...] = q
        pltpu.sync_copy(noise_v, y_hbm.at[pl.ds(wid * PPW, PPW)])
        pltpu.sync_copy(s_v, part_hbm.at[wid])
        pltpu.sync_copy(q_v, part_hbm.at[NW + wid])

    return k(c2, noise2, table, lw16, ct16)


def _tc_normalize(y2, part):
    blk = 512
    grid = (B // 2) // blk

    def body(part_ref, y_ref, o_ref):
        p = part_ref[...]
        s1 = jnp.sum(p[:NW, :])
        s2 = jnp.sum(p[NW:, :])
        mean = s1 / NTOT
        var = (s2 - s1 * s1 / NTOT) / (NTOT - 1)
        rstd = lax.rsqrt(var)
        o_ref[...] = (y_ref[...] - mean) * rstd

    return pl.pallas_call(
        body,
        grid=(grid,),
        in_specs=[
            pl.BlockSpec((2 * NW, 16), lambda i: (0, 0)),
            pl.BlockSpec((blk, 2 * D), lambda i: (i, 0)),
        ],
        out_specs=pl.BlockSpec((blk, 2 * D), lambda i: (i, 0)),
        out_shape=jax.ShapeDtypeStruct((B // 2, 2 * D), jnp.float32),
    )(part, y2)


def kernel(noise, c, embed_table, label_weight, c_training):
    c2 = c.reshape(NW, BPW)
    noise2 = noise.reshape(B // 2, 2 * D)
    lw16 = jnp.broadcast_to(label_weight.astype(jnp.float32), (16,))
    ct16 = jnp.broadcast_to(jnp.asarray(c_training, jnp.int32), (16,))
    y2, part = _sc_gather_stats(c2, noise2, embed_table.reshape(125000, 8, D), lw16, ct16)
    out2 = _tc_normalize(y2, part)
    return out2.reshape(B, D)


# speedup vs baseline: 1.0034x; 1.0034x over previous
import functools

import jax
import jax.numpy as jnp
from jax import lax
from jax.experimental import pallas as pl
from jax.experimental.pallas import tpu as pltpu
from jax.experimental.pallas import tpu_sc as plsc

B = 16384
D = 64
NW = 32
BPW = B // NW
PPW = BPW // 2
CHUNK = 16
NRING = 4
NCHUNK = BPW // CHUNK
NTOT = B * D


def _sc_gather_stats(c2, noise2, table, lw16, ct16):
    mesh = plsc.VectorSubcoreMesh(core_axis_name="c", subcore_axis_name="s")

    @functools.partial(
        pl.kernel,
        mesh=mesh,
        out_type=[
            jax.ShapeDtypeStruct((B // 2, 2 * D), jnp.float32),
            jax.ShapeDtypeStruct((2 * NW, 16), jnp.float32),
        ],
        scratch_types=[
            pltpu.VMEM((BPW,), jnp.int32),
            pltpu.VMEM((BPW + 16,), jnp.int32),
            pltpu.VMEM((BPW + 16,), jnp.int32),
            pltpu.VMEM((CHUNK, 8, D), jnp.float32),
            pltpu.VMEM((CHUNK, 8, D), jnp.float32),
            pltpu.VMEM((CHUNK, 8, D), jnp.float32),
            pltpu.VMEM((CHUNK, 8, D), jnp.float32),
            pltpu.VMEM((PPW, 2 * D), jnp.float32),
            pltpu.VMEM((16,), jnp.float32),
            pltpu.VMEM((16,), jnp.int32),
            pltpu.VMEM((16,), jnp.float32),
            pltpu.VMEM((16,), jnp.float32),
            pltpu.SemaphoreType.DMA,
            pltpu.SemaphoreType.DMA,
            pltpu.SemaphoreType.DMA,
            pltpu.SemaphoreType.DMA,
            pltpu.SemaphoreType.DMA,
        ],
    )
    def k(c_hbm, noise_hbm, table_hbm, lw_hbm, ct_hbm, y_hbm, part_hbm,
          raw_v, rb_v, row_v, buf0, buf1, buf2, buf3, noise_v, lw_v, ct_v,
          s_v, q_v, sem0, sem1, sem2, sem3, nsem):
        bufs = (buf0, buf1, buf2, buf3)
        sems = (sem0, sem1, sem2, sem3)
        wid = lax.axis_index("s") * 2 + lax.axis_index("c")
        pltpu.sync_copy(c_hbm.at[wid], raw_v)
        ncopy = pltpu.async_copy(noise_hbm.at[pl.ds(wid * PPW, PPW)], noise_v, nsem)
        for t in range(BPW // 16):
            v = raw_v[pl.ds(t * 16, 16)]
            rb_v[pl.ds(t * 16, 16)] = v >> 3
            row_v[pl.ds(t * 16, 16)] = v & 7

        def fire(k_idx, buf, sem):
            for g in range(0, CHUNK, 16):
                v = rb_v[pl.ds(k_idx * CHUNK + g, 16)]
                for l in range(16):
                    pltpu.async_copy(table_hbm.at[v[l]], buf.at[g + l], sem)

        def drain(buf, sem):
            pltpu.make_async_copy(table_hbm.at[pl.ds(0, CHUNK)], buf, sem).wait()

        pltpu.sync_copy(lw_hbm, lw_v)
        pltpu.sync_copy(ct_hbm, ct_v)
        lw = lw_v[...]
        ct = ct_v[...]
        scale = jnp.where(ct != 0, 1.0 / (1.0 + jnp.exp(-lw)), 0.0)

        for a in range(NRING - 1):
            fire(a, bufs[a], sems[a])
        ncopy.wait()
        s = jnp.zeros((16,), jnp.float32)
        q = jnp.zeros((16,), jnp.float32)
        for kk in range(NCHUNK):
            nxt = kk + NRING - 1
            if nxt < NCHUNK:
                fire(nxt, bufs[nxt % NRING], sems[nxt % NRING])
            drain(bufs[kk % NRING], sems[kk % NRING])
            buf = bufs[kk % NRING]

            def body(p, carry, kk=kk, buf=buf):
                s, q = carry
                rows = row_v[pl.ds(kk * CHUNK + 2 * p, 16)]
                for r in range(2):
                    row = rows[r]
                    for j in range(D // 16):
                        e = buf[2 * p + r, row, pl.ds(j * 16, 16)]
                        nz = noise_v[kk * (CHUNK // 2) + p, pl.ds(r * D + j * 16, 16)]
                        y = e * scale + nz
                        noise_v[kk * (CHUNK // 2) + p, pl.ds(r * D + j * 16, 16)] = y
                        s = s + y
                        q = q + y * y
                return s, q

            s, q = lax.fori_loop(0, CHUNK // 2, body, (s, q))

        s_v[...] = s
        q_v[...] = q
        pltpu.sync_copy(noise_v, y_hbm.at[pl.ds(wid * PPW, PPW)])
        pltpu.sync_copy(s_v, part_hbm.at[wid])
        pltpu.sync_copy(q_v, part_hbm.at[NW + wid])

    return k(c2, noise2, table, lw16, ct16)


def _tc_normalize(y2, part):
    blk = 512
    grid = (B // 2) // blk

    def body(part_ref, y_ref, o_ref):
        p = part_ref[...]
        s1 = jnp.sum(p[:NW, :])
        s2 = jnp.sum(p[NW:, :])
        mean = s1 / NTOT
        var = (s2 - s1 * s1 / NTOT) / (NTOT - 1)
        rstd = lax.rsqrt(var)
        o_ref[...] = (y_ref[...] - mean) * rstd

    return pl.pallas_call(
        body,
        grid=(grid,),
        in_specs=[
            pl.BlockSpec((2 * NW, 16), lambda i: (0, 0)),
            pl.BlockSpec((blk, 2 * D), lambda i: (i, 0)),
        ],
        out_specs=pl.BlockSpec((blk, 2 * D), lambda i: (i, 0)),
        out_shape=jax.ShapeDtypeStruct((B // 2, 2 * D), jnp.float32),
    )(part, y2)


def kernel(noise, c, embed_table, label_weight, c_training):
    c2 = c.reshape(NW, BPW)
    noise2 = noise.reshape(B // 2, 2 * D)
    lw16 = jnp.broadcast_to(label_weight.astype(jnp.float32), (16,))
    ct16 = jnp.broadcast_to(jnp.asarray(c_training, jnp.int32), (16,))
    y2, part = _sc_gather_stats(c2, noise2, embed_table.reshape(125000, 8, D), lw16, ct16)
    out2 = _tc_normalize(y2, part)
    return out2.reshape(B, D)
